# baseline (speedup 1.0000x reference)
import functools

import jax
import jax.numpy as jnp
from jax import lax
from jax.experimental import pallas as pl
from jax.experimental.pallas import tpu as pltpu


def _matmul_kernel(x_ref, w_ref, o_ref):
    x = x_ref[...].astype(jnp.bfloat16)
    o_ref[...] = lax.dot_general(
        x,
        w_ref[...],
        dimension_numbers=(((1,), (0,)), ((), ())),
        preferred_element_type=jnp.float32,
    )


@functools.partial(jax.jit, static_argnames=("tm", "n_split"))
def _my_linear(x2, w_t_bf16, tm, n_split):
    M, K = x2.shape
    N = w_t_bf16.shape[1]
    tn = N // n_split

    grid = (n_split, M // tm)

    cost = pl.CostEstimate(
        flops=2 * M * N * K,
        bytes_accessed=4 * M * K + 2 * N * K + 4 * M * N,
        transcendentals=0,
    )

    return pl.pallas_call(
        _matmul_kernel,
        out_shape=jax.ShapeDtypeStruct((M, N), jnp.float32),
        grid=grid,
        in_specs=[
            pl.BlockSpec((tm, K), lambda j, i: (i, 0)),
            pl.BlockSpec((K, tn), lambda j, i: (0, j)),
        ],
        out_specs=pl.BlockSpec((tm, tn), lambda j, i: (i, j)),
        compiler_params=pltpu.CompilerParams(
            dimension_semantics=("parallel", "arbitrary"),
            vmem_limit_bytes=60 * 1024 * 1024,
        ),
        cost_estimate=cost,
    )(x2, w_t_bf16)


def kernel(x, weight):
    orig_shape = x.shape
    K = orig_shape[-1]
    x2 = x.reshape(-1, K)
    N = weight.shape[0]
    w_t_bf16 = weight.astype(jnp.bfloat16).T
    out = _my_linear(x2, w_t_bf16, tm=256, n_split=2)
    return out.reshape(orig_shape[:-1] + (N,))


# speedup vs baseline: 1.0924x; 1.0924x over previous
import functools

import jax
import jax.numpy as jnp
from jax import lax
from jax.experimental import pallas as pl
from jax.experimental.pallas import tpu as pltpu


def _matmul_kernel(x_ref, w_ref, o_ref):
    x = x_ref[...].astype(jnp.bfloat16)
    o_ref[...] = lax.dot_general(
        x,
        w_ref[...],
        dimension_numbers=(((1,), (1,)), ((), ())),
        preferred_element_type=jnp.float32,
    )


@functools.partial(jax.jit, static_argnames=("tm", "n_split"))
def _my_linear(x2, w_bf16, tm, n_split):
    M, K = x2.shape
    N = w_bf16.shape[0]
    tn = N // n_split

    grid = (n_split, M // tm)

    cost = pl.CostEstimate(
        flops=2 * M * N * K,
        bytes_accessed=4 * M * K + 2 * N * K + 4 * M * N,
        transcendentals=0,
    )

    return pl.pallas_call(
        _matmul_kernel,
        out_shape=jax.ShapeDtypeStruct((M, N), jnp.float32),
        grid=grid,
        in_specs=[
            pl.BlockSpec((tm, K), lambda j, i: (i, 0)),
            pl.BlockSpec((tn, K), lambda j, i: (j, 0)),
        ],
        out_specs=pl.BlockSpec((tm, tn), lambda j, i: (i, j)),
        compiler_params=pltpu.CompilerParams(
            dimension_semantics=("parallel", "arbitrary"),
            vmem_limit_bytes=60 * 1024 * 1024,
        ),
        cost_estimate=cost,
    )(x2, w_bf16)


def kernel(x, weight):
    orig_shape = x.shape
    K = orig_shape[-1]
    x2 = x.reshape(-1, K)
    N = weight.shape[0]
    w_bf16 = weight.astype(jnp.bfloat16)
    out = _my_linear(x2, w_bf16, tm=512, n_split=2)
    return out.reshape(orig_shape[:-1] + (N,))
